# Initial kernel scaffold; baseline (speedup 1.0000x reference)
import jax
import jax.numpy as jnp
from jax.experimental import pallas as pl


def kernel(x, edge_index, W1, a_src1, a_dst1, eps1, W2, a_src2, a_dst2, eps2, c, W_out, b_out):
    raise NotImplementedError("write your pallas kernel here")


# speedup vs baseline: 3.6974x; 3.6974x over previous
import functools
import jax
import jax.numpy as jnp
from jax.experimental import pallas as pl
from jax.experimental.pallas import tpu as pltpu

f32 = jnp.float32
B = 1024
C = 1024


def _expmap(v, sqrt_c):
    norm = jnp.sqrt(jnp.sum(v * v, axis=-1, keepdims=True))
    norm = jnp.clip(norm, 1e-7, None)
    return jnp.tanh(sqrt_c * norm) * v / (sqrt_c * norm)


def _logmap(y, sqrt_c):
    norm = jnp.sqrt(jnp.sum(y * y, axis=-1, keepdims=True))
    norm = jnp.clip(norm, 1e-7, (1.0 - 1e-5) / sqrt_c)
    u = sqrt_c * norm
    atanh_u = 0.5 * jnp.log((1.0 + u) / (1.0 - u))
    return atanh_u * y / u


def _k0(x_ref, w1_ref, as_ref, ad_ref, par_ref, o_ref, *, hw):
    sc = jnp.sqrt(par_ref[0, 0])
    t = _logmap(_expmap(x_ref[...], sc), sc)
    h1 = jnp.dot(t, w1_ref[...], preferred_element_type=f32)
    o_ref[:, :hw] = h1
    o_ref[:, hw:hw + 8] = jnp.dot(h1, as_ref[...], preferred_element_type=f32)
    o_ref[:, hw + 8:hw + 16] = jnp.dot(h1, ad_ref[...], preferred_element_type=f32)


def _k1(src_ref, dst_ref, t1_ref, r8_ref, v_ref, accA, accB, *, nb_total, hw, n_edges):
    nb = pl.program_id(1)

    @pl.when(nb == 0)
    def _():
        accA[...] = jnp.zeros_like(accA)
        accB[...] = jnp.zeros_like(accB)

    col = jax.lax.broadcasted_iota(jnp.int32, (C, B), 1) + nb * B
    ohs = (src_ref[0] == col).astype(f32)
    ohd = (dst_ref[0] == col).astype(f32)
    tb = t1_ref[...]
    accA[...] += jnp.dot(ohs, tb[:, :hw + 8], preferred_element_type=f32)
    accB[...] += jnp.dot(ohd, tb[:, hw + 8:hw + 16], preferred_element_type=f32)

    @pl.when(nb == nb_total - 1)
    def _():
        a = accA[...]
        logit = a[:, hw:hw + 8] + accB[...]
        lr = jnp.where(logit >= 0, logit, 0.2 * logit)
        eid = pl.program_id(0) * C + jax.lax.broadcasted_iota(jnp.int32, (C, 8), 0)
        w = jnp.where(eid < n_edges, jnp.exp(lr), 0.0)
        wrep = jnp.dot(w, r8_ref[...], preferred_element_type=f32)
        v_ref[:, :hw] = a[:, :hw] * wrep
        v_ref[:, hw:hw + 8] = w


def _k2(dst_ref, v_ref, t1_ref, r8_ref, m8_ref, w2_ref, as2_ref, ad2_ref,
        par_ref, o_ref, acc, *, nc_total, hw, dh):
    db = pl.program_id(0)
    cb = pl.program_id(1)

    @pl.when(cb == 0)
    def _():
        acc[...] = jnp.zeros_like(acc)

    col = jax.lax.broadcasted_iota(jnp.int32, (C, B), 1) + db * B
    ohd = (dst_ref[0] == col).astype(f32)
    acc[...] += jax.lax.dot_general(ohd, v_ref[...], (((0,), (0,)), ((), ())),
                                    preferred_element_type=f32)

    @pl.when(cb == nc_total - 1)
    def _():
        a = acc[...]
        inv = 1.0 / (a[:, hw:hw + 8] + 1e-9)
        agg = a[:, :hw] * jnp.dot(inv, r8_ref[...], preferred_element_type=f32)
        z = (1.0 + par_ref[0, 1]) * t1_ref[:, :hw] + agg
        z = jnp.where(z > 0, z, jnp.exp(z) - 1.0)
        t1 = jnp.dot(z, m8_ref[...], preferred_element_type=f32)
        sc = jnp.sqrt(par_ref[0, 0])
        t1b = _logmap(_expmap(t1, sc), sc)
        h2 = jnp.dot(t1b, w2_ref[...], preferred_element_type=f32)
        o_ref[:, :dh] = h2
        o_ref[:, dh:dh + 1] = jnp.dot(h2, as2_ref[...], preferred_element_type=f32)
        o_ref[:, dh + 1:dh + 2] = jnp.dot(h2, ad2_ref[...], preferred_element_type=f32)


def _k3(src_ref, dst_ref, t2_ref, v_ref, accA, accB, *, nb_total, n_edges, dh):
    nb = pl.program_id(1)

    @pl.when(nb == 0)
    def _():
        accA[...] = jnp.zeros_like(accA)
        accB[...] = jnp.zeros_like(accB)

    col = jax.lax.broadcasted_iota(jnp.int32, (C, B), 1) + nb * B
    ohs = (src_ref[0] == col).astype(f32)
    ohd = (dst_ref[0] == col).astype(f32)
    tb = t2_ref[...]
    accA[...] += jnp.dot(ohs, tb[:, :dh + 1], preferred_element_type=f32)
    accB[...] += jnp.dot(ohd, tb[:, dh + 1:dh + 2], preferred_element_type=f32)

    @pl.when(nb == nb_total - 1)
    def _():
        a = accA[...]
        logit = a[:, dh:dh + 1] + accB[...]
        lr = jnp.where(logit >= 0, logit, 0.2 * logit)
        eid = pl.program_id(0) * C + jax.lax.broadcasted_iota(jnp.int32, (C, 1), 0)
        w = jnp.where(eid < n_edges, jnp.exp(lr), 0.0)
        v_ref[:, :dh] = a[:, :dh] * w
        v_ref[:, dh:dh + 1] = w


def _k4(dst_ref, v_ref, t2_ref, wout_ref, bout_ref, par_ref, o_ref, acc,
        *, nc_total, dh):
    db = pl.program_id(0)
    cb = pl.program_id(1)

    @pl.when(cb == 0)
    def _():
        acc[...] = jnp.zeros_like(acc)

    col = jax.lax.broadcasted_iota(jnp.int32, (C, B), 1) + db * B
    ohd = (dst_ref[0] == col).astype(f32)
    acc[...] += jax.lax.dot_general(ohd, v_ref[...], (((0,), (0,)), ((), ())),
                                    preferred_element_type=f32)

    @pl.when(cb == nc_total - 1)
    def _():
        a = acc[...]
        agg = a[:, :dh] / (a[:, dh:dh + 1] + 1e-9)
        z = (1.0 + par_ref[0, 2]) * t2_ref[:, :dh] + agg
        z = jnp.where(z > 0, z, jnp.exp(z) - 1.0)
        sc = jnp.sqrt(par_ref[0, 0])
        euclid = _logmap(_expmap(z, sc), sc)
        o_ref[...] = jnp.dot(euclid, wout_ref[...], preferred_element_type=f32) \
            + bout_ref[...]


def kernel(x, edge_index, W1, a_src1, a_dst1, eps1, W2, a_src2, a_dst2, eps2,
           c, W_out, b_out):
    n, d_in = x.shape
    n_edges = edge_index.shape[1]
    h1_heads, _, d_hid = W1.shape
    n_cls = W_out.shape[1]
    hw = h1_heads * d_hid

    nb_total = -(-n // B)
    np_ = nb_total * B
    nc_total = -(-n_edges // C)
    ep = nc_total * C

    xp = jnp.pad(x.astype(f32), ((0, np_ - n), (0, 0)))
    src = jnp.pad(edge_index[0].astype(jnp.int32), (0, ep - n_edges))
    dst = jnp.pad(edge_index[1].astype(jnp.int32), (0, ep - n_edges))
    src_r = src.reshape(nc_total, C, 1)
    dst_r = dst.reshape(nc_total, C, 1)

    W1f = W1.astype(f32).transpose(1, 0, 2).reshape(d_in, hw)
    eye_h = jnp.eye(h1_heads, dtype=f32)
    As1 = (eye_h[:, None, :] * a_src1.astype(f32)[:, :, None]).reshape(hw, h1_heads)
    Ad1 = (eye_h[:, None, :] * a_dst1.astype(f32)[:, :, None]).reshape(hw, h1_heads)
    R8 = (eye_h[:, :, None] * jnp.ones((1, 1, d_hid), f32)).reshape(h1_heads, hw)
    M8 = (jnp.ones((h1_heads, 1, 1), f32) * jnp.eye(d_hid, dtype=f32)[None]
          ).reshape(hw, d_hid) / h1_heads
    W2f = W2.astype(f32).transpose(1, 0, 2).reshape(d_hid, d_hid)
    As2 = a_src2.astype(f32).reshape(d_hid, 1)
    Ad2 = a_dst2.astype(f32).reshape(d_hid, 1)
    cc = jnp.clip(c.astype(f32), 1e-3, None)
    par = jnp.stack([cc, eps1.astype(f32), eps2.astype(f32)]).reshape(1, 3)
    bout2 = b_out.astype(f32).reshape(1, n_cls)

    full = lambda shape: pl.BlockSpec(shape, lambda *_: tuple(0 for _ in shape))

    t1 = pl.pallas_call(
        functools.partial(_k0, hw=hw),
        grid=(nb_total,),
        in_specs=[pl.BlockSpec((B, d_in), lambda i: (i, 0)),
                  full((d_in, hw)), full((hw, h1_heads)), full((hw, h1_heads)),
                  full((1, 3))],
        out_specs=pl.BlockSpec((B, hw + 16), lambda i: (i, 0)),
        out_shape=jax.ShapeDtypeStruct((np_, hw + 16), f32),
    )(xp, W1f, As1, Ad1, par)

    v1 = pl.pallas_call(
        functools.partial(_k1, nb_total=nb_total, hw=hw, n_edges=n_edges),
        grid=(nc_total, nb_total),
        in_specs=[pl.BlockSpec((1, C, 1), lambda cb, nb: (cb, 0, 0)),
                  pl.BlockSpec((1, C, 1), lambda cb, nb: (cb, 0, 0)),
                  pl.BlockSpec((B, hw + 16), lambda cb, nb: (nb, 0)),
                  full((h1_heads, hw))],
        out_specs=pl.BlockSpec((C, hw + 8), lambda cb, nb: (cb, 0)),
        out_shape=jax.ShapeDtypeStruct((ep, hw + 8), f32),
        scratch_shapes=[pltpu.VMEM((C, hw + 8), f32),
                        pltpu.VMEM((C, h1_heads), f32)],
    )(src_r, dst_r, t1, R8)

    t2 = pl.pallas_call(
        functools.partial(_k2, nc_total=nc_total, hw=hw, dh=d_hid),
        grid=(nb_total, nc_total),
        in_specs=[pl.BlockSpec((1, C, 1), lambda db, cb: (cb, 0, 0)),
                  pl.BlockSpec((C, hw + 8), lambda db, cb: (cb, 0)),
                  pl.BlockSpec((B, hw + 16), lambda db, cb: (db, 0)),
                  full((h1_heads, hw)), full((hw, d_hid)), full((d_hid, d_hid)),
                  full((d_hid, 1)), full((d_hid, 1)), full((1, 3))],
        out_specs=pl.BlockSpec((B, d_hid + 2), lambda db, cb: (db, 0)),
        out_shape=jax.ShapeDtypeStruct((np_, d_hid + 2), f32),
        scratch_shapes=[pltpu.VMEM((B, hw + 8), f32)],
    )(dst_r, v1, t1, R8, M8, W2f, As2, Ad2, par)

    v2 = pl.pallas_call(
        functools.partial(_k3, nb_total=nb_total, n_edges=n_edges, dh=d_hid),
        grid=(nc_total, nb_total),
        in_specs=[pl.BlockSpec((1, C, 1), lambda cb, nb: (cb, 0, 0)),
                  pl.BlockSpec((1, C, 1), lambda cb, nb: (cb, 0, 0)),
                  pl.BlockSpec((B, d_hid + 2), lambda cb, nb: (nb, 0))],
        out_specs=pl.BlockSpec((C, d_hid + 1), lambda cb, nb: (cb, 0)),
        out_shape=jax.ShapeDtypeStruct((ep, d_hid + 1), f32),
        scratch_shapes=[pltpu.VMEM((C, d_hid + 1), f32),
                        pltpu.VMEM((C, 1), f32)],
    )(src_r, dst_r, t2)

    out = pl.pallas_call(
        functools.partial(_k4, nc_total=nc_total, dh=d_hid),
        grid=(nb_total, nc_total),
        in_specs=[pl.BlockSpec((1, C, 1), lambda db, cb: (cb, 0, 0)),
                  pl.BlockSpec((C, d_hid + 1), lambda db, cb: (cb, 0)),
                  pl.BlockSpec((B, d_hid + 2), lambda db, cb: (db, 0)),
                  full((d_hid, n_cls)), full((1, n_cls)), full((1, 3))],
        out_specs=pl.BlockSpec((B, n_cls), lambda db, cb: (db, 0)),
        out_shape=jax.ShapeDtypeStruct((np_, n_cls), f32),
        scratch_shapes=[pltpu.VMEM((B, d_hid + 1), f32)],
    )(dst_r, v2, t2, W_out.astype(f32), bout2, par)

    return out[:n]
